# baseline (speedup 1.0000x reference)
import functools

import jax
import jax.numpy as jnp
from jax import lax
from jax.experimental import pallas as pl
from jax.experimental.pallas import tpu as pltpu
from jax.experimental.pallas import tpu_sc as plsc

_SMOOTH = 1e-06
_NB_SC = 2
_NTILES = 32
_QSEC = 384
_SEC = 3 * _QSEC


def _tc_body(nB, C, pred_ref, tgt_ref, out_ref, dsum_ref):
    b = pl.program_id(0)

    @pl.when(b == 0)
    def _init_scalar():
        dsum_ref[0] = 0.0

    H = tgt_ref.shape[1]

    def _tree128(x):
        return (x[:, 0:128] + x[:, 128:256]) + (x[:, 256:384] + x[:, 384:512])

    zi = jnp.zeros((8, 128), jnp.float32)
    total = dsum_ref[0]
    for c in range(C):
        ai = zi
        ac = zi
        asum = zi
        for k in range(H // 8):
            tk = tgt_ref[0, k * 8:(k + 1) * 8, :]
            pk = pred_ref[0, c, k * 8:(k + 1) * 8, :]
            m = tk == c
            ai = ai + _tree128(jnp.where(m, pk, 0.0))
            ac = ac + _tree128(jnp.where(m, 1.0, 0.0))
            asum = asum + _tree128(pk)
        inter = jnp.sum(ai)
        cnt = jnp.sum(ac)
        psum = jnp.sum(asum)
        total += (2.0 * inter + _SMOOTH) / (psum + cnt + _SMOOTH)
    dsum_ref[0] = total

    @pl.when(b == nB - 1)
    def _emit():
        out_ref[0] = dsum_ref[0]


def _sc_body(B, C, HW, pflat_ref, tflat_ref, out_ref, tbuf, pbuf, row_ref):
    ncores = 2
    wid = lax.axis_index("s") * ncores + lax.axis_index("c")
    npx = HW // _NTILES
    px0 = wid * npx
    nvec = npx // 16

    z16 = jnp.zeros((16,), jnp.float32)
    for j in range(_SEC * _NB_SC // 16):
        row_ref[pl.ds(j * 16, 16)] = z16

    for bb in range(_NB_SC):
        b = B - _NB_SC + bb
        pltpu.sync_copy(tflat_ref.at[b, pl.ds(px0, npx)], tbuf)
        for c in range(C):
            pltpu.sync_copy(pflat_ref.at[b * C + c, pl.ds(px0, npx)], pbuf)

            def _vbody(i, carry):
                ai, ac, asum = carry
                t = tbuf[pl.ds(i * 16, 16)]
                p = pbuf[pl.ds(i * 16, 16)]
                m = t == c
                return (ai + jnp.where(m, p, 0.0),
                        ac + jnp.where(m, 1.0, 0.0),
                        asum + p)

            ai, ac, asum = lax.fori_loop(0, nvec, _vbody, (z16, z16, z16))
            base = bb * _SEC + c * 16
            row_ref[pl.ds(base, 16)] = ai
            row_ref[pl.ds(base + _QSEC, 16)] = asum
            row_ref[pl.ds(base + 2 * _QSEC, 16)] = ac

    pltpu.sync_copy(row_ref, out_ref.at[wid])


def _combine_body(B, C, sc_ref, w_ref, dsum_ref, out_ref):
    x = sc_ref[...]
    r = ((x[0:8] + x[8:16]) + (x[16:24] + x[24:32]))
    r = r[0:4] + r[4:8]
    r = r[0:2] + r[2:4]
    r = r[0:1] + r[1:2]
    w = w_ref[...]
    lane = lax.broadcasted_iota(jnp.int32, (1, 128), 1)
    total = dsum_ref[0]
    for bb in range(_NB_SC):
        base = bb * _SEC
        inter = jnp.dot(r[:, base:base + _QSEC], w,
                        preferred_element_type=jnp.float32)
        psum = jnp.dot(r[:, base + _QSEC:base + 2 * _QSEC], w,
                       preferred_element_type=jnp.float32)
        cnt = jnp.dot(r[:, base + 2 * _QSEC:base + 3 * _QSEC], w,
                      preferred_element_type=jnp.float32)
        dice = (2.0 * inter + _SMOOTH) / (psum + cnt + _SMOOTH)
        dice = jnp.where(lane < C, dice, 0.0)
        total += jnp.sum(dice)
    out_ref[0] = 1.0 - total / (B * C)


def kernel(pred, target):
    B, C, H, W = pred.shape
    HW = H * W
    nB = B - _NB_SC

    pred_flat = pred.reshape(B * C, HW)
    target_flat = target.reshape(B, HW)

    mesh = plsc.VectorSubcoreMesh(core_axis_name="c", subcore_axis_name="s")
    sc_kernel = functools.partial(
        pl.kernel,
        mesh=mesh,
        out_type=jax.ShapeDtypeStruct((_NTILES, _SEC * _NB_SC), jnp.float32),
        scratch_types=[
            pltpu.VMEM((HW // _NTILES,), jnp.int32),
            pltpu.VMEM((HW // _NTILES,), jnp.float32),
            pltpu.VMEM((_SEC * _NB_SC,), jnp.float32),
        ],
    )(functools.partial(_sc_body, B, C, HW))
    sc_out = sc_kernel(pred_flat, target_flat)

    tc_dsum = pl.pallas_call(
        functools.partial(_tc_body, nB, C),
        grid=(nB,),
        in_specs=[
            pl.BlockSpec((1, C, H, W), lambda b: (b, 0, 0, 0)),
            pl.BlockSpec((1, H, W), lambda b: (b, 0, 0)),
        ],
        out_specs=pl.BlockSpec(memory_space=pltpu.SMEM),
        out_shape=jax.ShapeDtypeStruct((1,), jnp.float32),
        scratch_shapes=[
            pltpu.SMEM((1,), jnp.float32),
        ],
        compiler_params=pltpu.CompilerParams(
            dimension_semantics=("arbitrary",)),
    )(pred, target)

    wmat = (jnp.arange(_QSEC)[:, None] // 16
            == jnp.arange(128)[None, :]).astype(jnp.float32)

    loss = pl.pallas_call(
        functools.partial(_combine_body, B, C),
        in_specs=[
            pl.BlockSpec((_NTILES, _SEC * _NB_SC), lambda: (0, 0)),
            pl.BlockSpec((_QSEC, 128), lambda: (0, 0)),
            pl.BlockSpec(memory_space=pltpu.SMEM),
        ],
        out_specs=pl.BlockSpec(memory_space=pltpu.SMEM),
        out_shape=jax.ShapeDtypeStruct((1,), jnp.float32),
    )(sc_out, wmat, tc_dsum)
    return loss[0]


# speedup vs baseline: 3.1595x; 3.1595x over previous
import functools

import jax
import jax.numpy as jnp
from jax import lax
from jax.experimental import pallas as pl
from jax.experimental.pallas import tpu as pltpu
from jax.experimental.pallas import tpu_sc as plsc

_SMOOTH = 1e-06
_NB_SC = 2
_NTILES = 32
_QSEC = 384
_SEC = 3 * _QSEC


def _tc_body(nB, C, pred_ref, tgt_ref, out_ref, dsum_ref):
    b = pl.program_id(0)

    @pl.when(b == 0)
    def _init_scalar():
        dsum_ref[0] = 0.0

    H = tgt_ref.shape[1]

    def _tree128(x):
        return (x[:, 0:128] + x[:, 128:256]) + (x[:, 256:384] + x[:, 384:512])

    zi = jnp.zeros((8, 128), jnp.float32)
    total = dsum_ref[0]
    for c in range(C):
        ai = zi
        ac = zi
        asum = zi
        for k in range(H // 8):
            tk = tgt_ref[0, k * 8:(k + 1) * 8, :]
            pk = pred_ref[0, c, k * 8:(k + 1) * 8, :]
            m = tk == c
            ai = ai + _tree128(jnp.where(m, pk, 0.0))
            ac = ac + _tree128(jnp.where(m, 1.0, 0.0))
            asum = asum + _tree128(pk)
        inter = jnp.sum(ai)
        cnt = jnp.sum(ac)
        psum = jnp.sum(asum)
        total += (2.0 * inter + _SMOOTH) / (psum + cnt + _SMOOTH)
    dsum_ref[0] = total

    @pl.when(b == nB - 1)
    def _emit():
        out_ref[0] = dsum_ref[0]


def _sc_body(B, C, H, W, pred_ref, tgt_ref, out_ref, tbuf, pbuf0, pbuf1,
             row_ref, sem0, sem1):
    ncores = 2
    wid = lax.axis_index("s") * ncores + lax.axis_index("c")
    nrow = H // _NTILES
    r0 = wid * nrow

    z16 = jnp.zeros((16,), jnp.float32)
    for j in range(_SEC * _NB_SC // 16):
        row_ref[pl.ds(j * 16, 16)] = z16

    pbufs = (pbuf0, pbuf1)
    sems = (sem0, sem1)
    for bb in range(_NB_SC):
        b = B - _NB_SC + bb
        pltpu.sync_copy(tgt_ref.at[b, pl.ds(r0, nrow), :], tbuf)
        cps = {0: pltpu.async_copy(
            pred_ref.at[b, 0, pl.ds(r0, nrow), :], pbuf0, sem0)}
        for c in range(C):
            cps.pop(c % 2).wait()
            if c + 1 < C:
                cps[(c + 1) % 2] = pltpu.async_copy(
                    pred_ref.at[b, c + 1, pl.ds(r0, nrow), :],
                    pbufs[(c + 1) % 2], sems[(c + 1) % 2])
            pbuf = pbufs[c % 2]

            def _row_body(r, carry):
                def _grp_body(v, carry2):
                    accs = list(carry2)
                    for j in range(4):
                        sl = pl.ds((v * 4 + j) * 16, 16)
                        t = tbuf[r, sl]
                        p = pbuf[r, sl]
                        m = t == c
                        accs[3 * j] = accs[3 * j] + jnp.where(m, p, 0.0)
                        accs[3 * j + 1] = accs[3 * j + 1] + jnp.where(
                            m, 1.0, 0.0)
                        accs[3 * j + 2] = accs[3 * j + 2] + p
                    return tuple(accs)

                return lax.fori_loop(0, W // 64, _grp_body, carry)

            accs = lax.fori_loop(0, nrow, _row_body, (z16,) * 12)
            ai = (accs[0] + accs[3]) + (accs[6] + accs[9])
            ac = (accs[1] + accs[4]) + (accs[7] + accs[10])
            asum = (accs[2] + accs[5]) + (accs[8] + accs[11])
            base = bb * _SEC + c * 16
            row_ref[pl.ds(base, 16)] = ai
            row_ref[pl.ds(base + _QSEC, 16)] = asum
            row_ref[pl.ds(base + 2 * _QSEC, 16)] = ac

    pltpu.sync_copy(row_ref, out_ref.at[wid])


def _combine_body(B, C, sc_ref, w_ref, dsum_ref, out_ref):
    x = sc_ref[...]
    r = ((x[0:8] + x[8:16]) + (x[16:24] + x[24:32]))
    r = r[0:4] + r[4:8]
    r = r[0:2] + r[2:4]
    r = r[0:1] + r[1:2]
    w = w_ref[...]
    lane = lax.broadcasted_iota(jnp.int32, (1, 128), 1)
    total = dsum_ref[0]
    for bb in range(_NB_SC):
        base = bb * _SEC
        inter = jnp.dot(r[:, base:base + _QSEC], w,
                        preferred_element_type=jnp.float32)
        psum = jnp.dot(r[:, base + _QSEC:base + 2 * _QSEC], w,
                       preferred_element_type=jnp.float32)
        cnt = jnp.dot(r[:, base + 2 * _QSEC:base + 3 * _QSEC], w,
                      preferred_element_type=jnp.float32)
        dice = (2.0 * inter + _SMOOTH) / (psum + cnt + _SMOOTH)
        dice = jnp.where(lane < C, dice, 0.0)
        total += jnp.sum(dice)
    out_ref[0] = 1.0 - total / (B * C)


def kernel(pred, target):
    B, C, H, W = pred.shape
    HW = H * W
    nB = B - _NB_SC

    nrow = H // _NTILES
    mesh = plsc.VectorSubcoreMesh(core_axis_name="c", subcore_axis_name="s")
    sc_kernel = functools.partial(
        pl.kernel,
        mesh=mesh,
        out_type=jax.ShapeDtypeStruct((_NTILES, _SEC * _NB_SC), jnp.float32),
        scratch_types=[
            pltpu.VMEM((nrow, W), jnp.int32),
            pltpu.VMEM((nrow, W), jnp.float32),
            pltpu.VMEM((nrow, W), jnp.float32),
            pltpu.VMEM((_SEC * _NB_SC,), jnp.float32),
            pltpu.SemaphoreType.DMA,
            pltpu.SemaphoreType.DMA,
        ],
    )(functools.partial(_sc_body, B, C, H, W))
    sc_out = sc_kernel(pred, target)

    tc_dsum = pl.pallas_call(
        functools.partial(_tc_body, nB, C),
        grid=(nB,),
        in_specs=[
            pl.BlockSpec((1, C, H, W), lambda b: (b, 0, 0, 0)),
            pl.BlockSpec((1, H, W), lambda b: (b, 0, 0)),
        ],
        out_specs=pl.BlockSpec(memory_space=pltpu.SMEM),
        out_shape=jax.ShapeDtypeStruct((1,), jnp.float32),
        scratch_shapes=[
            pltpu.SMEM((1,), jnp.float32),
        ],
        compiler_params=pltpu.CompilerParams(
            dimension_semantics=("arbitrary",)),
    )(pred, target)

    wmat = (jnp.arange(_QSEC)[:, None] // 16
            == jnp.arange(128)[None, :]).astype(jnp.float32)

    loss = pl.pallas_call(
        functools.partial(_combine_body, B, C),
        in_specs=[
            pl.BlockSpec((_NTILES, _SEC * _NB_SC), lambda: (0, 0)),
            pl.BlockSpec((_QSEC, 128), lambda: (0, 0)),
            pl.BlockSpec(memory_space=pltpu.SMEM),
        ],
        out_specs=pl.BlockSpec(memory_space=pltpu.SMEM),
        out_shape=jax.ShapeDtypeStruct((1,), jnp.float32),
    )(sc_out, wmat, tc_dsum)
    return loss[0]


# speedup vs baseline: 3.5092x; 1.1107x over previous
import functools

import jax
import jax.numpy as jnp
from jax import lax
from jax.experimental import pallas as pl
from jax.experimental.pallas import tpu as pltpu
from jax.experimental.pallas import tpu_sc as plsc

_SMOOTH = 1e-06
_NB_SC = 1
_NTILES = 32
_QSEC = 384
_SEC = 3 * _QSEC


def _tc_body(nB, C, pred_ref, tgt_ref, out_ref, dsum_ref):
    b = pl.program_id(0)

    @pl.when(b == 0)
    def _init_scalar():
        dsum_ref[0] = 0.0

    H = tgt_ref.shape[1]

    def _tree128(x):
        return (x[:, 0:128] + x[:, 128:256]) + (x[:, 256:384] + x[:, 384:512])

    zi = jnp.zeros((8, 128), jnp.float32)
    total = dsum_ref[0]
    for c in range(C):
        ai = zi
        ac = zi
        asum = zi
        for k in range(H // 8):
            tk = tgt_ref[0, k * 8:(k + 1) * 8, :]
            pk = pred_ref[0, c, k * 8:(k + 1) * 8, :]
            m = tk == c
            ai = ai + _tree128(jnp.where(m, pk, 0.0))
            ac = ac + _tree128(jnp.where(m, 1.0, 0.0))
            asum = asum + _tree128(pk)
        inter = jnp.sum(ai)
        cnt = jnp.sum(ac)
        psum = jnp.sum(asum)
        total += (2.0 * inter + _SMOOTH) / (psum + cnt + _SMOOTH)
    dsum_ref[0] = total

    @pl.when(b == nB - 1)
    def _emit():
        out_ref[0] = dsum_ref[0]


def _sc_body(B, C, H, W, pred_ref, tgt_ref, out_ref, tbuf, pbuf0, pbuf1,
             row_ref, sem0, sem1):
    ncores = 2
    wid = lax.axis_index("s") * ncores + lax.axis_index("c")
    nrow = H // _NTILES
    r0 = wid * nrow

    z16 = jnp.zeros((16,), jnp.float32)
    for j in range(_SEC * _NB_SC // 16):
        row_ref[pl.ds(j * 16, 16)] = z16

    pbufs = (pbuf0, pbuf1)
    sems = (sem0, sem1)
    for bb in range(_NB_SC):
        b = B - _NB_SC + bb
        pltpu.sync_copy(tgt_ref.at[b, pl.ds(r0, nrow), :], tbuf)
        cps = {0: pltpu.async_copy(
            pred_ref.at[b, 0, pl.ds(r0, nrow), :], pbuf0, sem0)}
        for c in range(C):
            cps.pop(c % 2).wait()
            if c + 1 < C:
                cps[(c + 1) % 2] = pltpu.async_copy(
                    pred_ref.at[b, c + 1, pl.ds(r0, nrow), :],
                    pbufs[(c + 1) % 2], sems[(c + 1) % 2])
            pbuf = pbufs[c % 2]

            def _row_body(r, carry):
                def _grp_body(v, carry2):
                    accs = list(carry2)
                    for j in range(4):
                        sl = pl.ds((v * 4 + j) * 16, 16)
                        t = tbuf[r, sl]
                        p = pbuf[r, sl]
                        m = t == c
                        accs[3 * j] = accs[3 * j] + jnp.where(m, p, 0.0)
                        accs[3 * j + 1] = accs[3 * j + 1] + jnp.where(
                            m, 1.0, 0.0)
                        accs[3 * j + 2] = accs[3 * j + 2] + p
                    return tuple(accs)

                return lax.fori_loop(0, W // 64, _grp_body, carry)

            accs = lax.fori_loop(0, nrow, _row_body, (z16,) * 12)
            ai = (accs[0] + accs[3]) + (accs[6] + accs[9])
            ac = (accs[1] + accs[4]) + (accs[7] + accs[10])
            asum = (accs[2] + accs[5]) + (accs[8] + accs[11])
            base = bb * _SEC + c * 16
            row_ref[pl.ds(base, 16)] = ai
            row_ref[pl.ds(base + _QSEC, 16)] = asum
            row_ref[pl.ds(base + 2 * _QSEC, 16)] = ac

    pltpu.sync_copy(row_ref, out_ref.at[wid])


def _combine_body(B, C, sc_ref, w_ref, dsum_ref, out_ref):
    x = sc_ref[...]
    r = ((x[0:8] + x[8:16]) + (x[16:24] + x[24:32]))
    r = r[0:4] + r[4:8]
    r = r[0:2] + r[2:4]
    r = r[0:1] + r[1:2]
    w = w_ref[...]
    lane = lax.broadcasted_iota(jnp.int32, (1, 128), 1)
    total = dsum_ref[0]
    for bb in range(_NB_SC):
        base = bb * _SEC
        inter = jnp.dot(r[:, base:base + _QSEC], w,
                        preferred_element_type=jnp.float32)
        psum = jnp.dot(r[:, base + _QSEC:base + 2 * _QSEC], w,
                       preferred_element_type=jnp.float32)
        cnt = jnp.dot(r[:, base + 2 * _QSEC:base + 3 * _QSEC], w,
                      preferred_element_type=jnp.float32)
        dice = (2.0 * inter + _SMOOTH) / (psum + cnt + _SMOOTH)
        dice = jnp.where(lane < C, dice, 0.0)
        total += jnp.sum(dice)
    out_ref[0] = 1.0 - total / (B * C)


def kernel(pred, target):
    B, C, H, W = pred.shape
    HW = H * W
    nB = B - _NB_SC

    nrow = H // _NTILES
    mesh = plsc.VectorSubcoreMesh(core_axis_name="c", subcore_axis_name="s")
    sc_kernel = functools.partial(
        pl.kernel,
        mesh=mesh,
        out_type=jax.ShapeDtypeStruct((_NTILES, _SEC * _NB_SC), jnp.float32),
        scratch_types=[
            pltpu.VMEM((nrow, W), jnp.int32),
            pltpu.VMEM((nrow, W), jnp.float32),
            pltpu.VMEM((nrow, W), jnp.float32),
            pltpu.VMEM((_SEC * _NB_SC,), jnp.float32),
            pltpu.SemaphoreType.DMA,
            pltpu.SemaphoreType.DMA,
        ],
    )(functools.partial(_sc_body, B, C, H, W))
    sc_out = sc_kernel(pred, target)

    tc_dsum = pl.pallas_call(
        functools.partial(_tc_body, nB, C),
        grid=(nB,),
        in_specs=[
            pl.BlockSpec((1, C, H, W), lambda b: (b, 0, 0, 0)),
            pl.BlockSpec((1, H, W), lambda b: (b, 0, 0)),
        ],
        out_specs=pl.BlockSpec(memory_space=pltpu.SMEM),
        out_shape=jax.ShapeDtypeStruct((1,), jnp.float32),
        scratch_shapes=[
            pltpu.SMEM((1,), jnp.float32),
        ],
        compiler_params=pltpu.CompilerParams(
            dimension_semantics=("arbitrary",)),
    )(pred, target)

    wmat = (jnp.arange(_QSEC)[:, None] // 16
            == jnp.arange(128)[None, :]).astype(jnp.float32)

    loss = pl.pallas_call(
        functools.partial(_combine_body, B, C),
        in_specs=[
            pl.BlockSpec((_NTILES, _SEC * _NB_SC), lambda: (0, 0)),
            pl.BlockSpec((_QSEC, 128), lambda: (0, 0)),
            pl.BlockSpec(memory_space=pltpu.SMEM),
        ],
        out_specs=pl.BlockSpec(memory_space=pltpu.SMEM),
        out_shape=jax.ShapeDtypeStruct((1,), jnp.float32),
    )(sc_out, wmat, tc_dsum)
    return loss[0]


# speedup vs baseline: 4.7677x; 1.3586x over previous
import functools

import jax
import jax.numpy as jnp
from jax.experimental import pallas as pl
from jax.experimental.pallas import tpu as pltpu

_SMOOTH = 1e-06


def _dice_body(B, C, pred_ref, tgt_ref, out_ref, acc_i, acc_s, acc_c,
               dsum_ref):
    b = pl.program_id(0)

    @pl.when(b == 0)
    def _init_scalar():
        dsum_ref[0] = 0.0

    H = tgt_ref.shape[1]

    def _tree128(x):
        return (x[:, 0:128] + x[:, 128:256]) + (x[:, 256:384] + x[:, 384:512])

    zi = jnp.zeros((8, 128), jnp.float32)
    for c in range(C):
        ai = zi
        ac = zi
        asum = zi
        for k in range(H // 8):
            tk = tgt_ref[0, k * 8:(k + 1) * 8, :]
            pk = pred_ref[0, c, k * 8:(k + 1) * 8, :]
            m = tk == c
            ai = ai + _tree128(jnp.where(m, pk, 0.0))
            ac = ac + _tree128(jnp.where(m, 1.0, 0.0))
            asum = asum + _tree128(pk)
        acc_i[c, :, :] = ai
        acc_c[c, :, :] = ac
        acc_s[c, :, :] = asum

    total = dsum_ref[0]
    for c in range(C):
        inter = jnp.sum(acc_i[c, :, :])
        cnt = jnp.sum(acc_c[c, :, :])
        psum = jnp.sum(acc_s[c, :, :])
        total += (2.0 * inter + _SMOOTH) / (psum + cnt + _SMOOTH)
    dsum_ref[0] = total

    @pl.when(b == B - 1)
    def _emit():
        out_ref[0] = 1.0 - dsum_ref[0] / (B * C)


def kernel(pred, target):
    B, C, H, W = pred.shape

    body = functools.partial(_dice_body, B, C)

    out = pl.pallas_call(
        body,
        grid=(B,),
        in_specs=[
            pl.BlockSpec((1, C, H, W), lambda b: (b, 0, 0, 0)),
            pl.BlockSpec((1, H, W), lambda b: (b, 0, 0)),
        ],
        out_specs=pl.BlockSpec(memory_space=pltpu.SMEM),
        out_shape=jax.ShapeDtypeStruct((1,), jnp.float32),
        scratch_shapes=[
            pltpu.VMEM((C, 8, 128), jnp.float32),
            pltpu.VMEM((C, 8, 128), jnp.float32),
            pltpu.VMEM((C, 8, 128), jnp.float32),
            pltpu.SMEM((1,), jnp.float32),
        ],
        compiler_params=pltpu.CompilerParams(
            dimension_semantics=("arbitrary",)),
    )(pred, target)
    return out[0]
